# baseline (speedup 1.0000x reference)
import jax
import jax.numpy as jnp
from jax.experimental import pallas as pl
from jax.experimental.pallas import tpu as pltpu

_N = 10000
_TM1 = 400
_TM = 800
_NBLK = 13
_NR = 10400
_F32 = jnp.float32
_BF16 = jnp.bfloat16
_PARAMS = pltpu.CompilerParams(vmem_limit_bytes=120 * 1024 * 1024)

_STREAM_SPEC = pl.BlockSpec(
    (_TM, _N), lambda i: (i, 0), pipeline_mode=pl.Buffered(buffer_count=3)
)


def _lrelu(y):
    return jnp.where(y > 0, y, 0.01 * y)


def _sup1_body(x_ref, w_ref, o_ref):
    o_ref[...] = jnp.dot(
        x_ref[...], w_ref[...], preferred_element_type=_F32
    ).astype(_BF16)


def _layer1_body(a_ref, s_ref, w_ref, b_ref, a16_ref, sup_ref):
    a16 = a_ref[...].astype(_BF16)
    a16_ref[...] = a16
    y = jnp.dot(a16, s_ref[...], preferred_element_type=_F32) + b_ref[...]
    h = _lrelu(y)
    sup_ref[...] = jnp.dot(h, w_ref[...], preferred_element_type=_F32).astype(_BF16)


def _deep_body(adj_ref, s2_ref, w3_ref, w4_ref, w5_ref, w6_ref, w7_ref, w8_ref,
               b2_ref, b3_ref, b4_ref, b5_ref, b6_ref, b7_ref, b8_ref,
               lat_ref, out_ref, supa_ref, supb_ref, cnt_ref):

    def run_layer(step, out_specs=(), out_refs=()):
        cnt_ref[0] = 0

        def inner(a_ref, *orefs):
            i = cnt_ref[0]
            cnt_ref[0] = i + 1
            step(a_ref[...], pl.ds(i * _TM, _TM), *orefs)

        pltpu.emit_pipeline(
            inner, grid=(_NBLK,), in_specs=[_STREAM_SPEC],
            out_specs=list(out_specs),
        )(adj_ref, *out_refs)

    def l2(a, rows):
        h = _lrelu(jnp.dot(a, s2_ref[...], preferred_element_type=_F32) + b2_ref[...])
        supa_ref[rows, :16] = jnp.dot(h, w3_ref[...], preferred_element_type=_F32).astype(_BF16)

    def l3(a, rows):
        h = _lrelu(jnp.dot(a, supa_ref[:_N, :16], preferred_element_type=_F32) + b3_ref[...])
        supb_ref[rows, :8] = jnp.dot(h, w4_ref[...], preferred_element_type=_F32).astype(_BF16)

    def l4(a, rows, lat_blk_ref):
        y = jnp.dot(a, supb_ref[:_N, :8], preferred_element_type=_F32) + b4_ref[...]
        lat_blk_ref[...] = y
        supa_ref[rows, :16] = jnp.dot(y, w5_ref[...], preferred_element_type=_F32).astype(_BF16)

    def l5(a, rows):
        h = _lrelu(jnp.dot(a, supa_ref[:_N, :16], preferred_element_type=_F32) + b5_ref[...])
        supb_ref[rows, :32] = jnp.dot(h, w6_ref[...], preferred_element_type=_F32).astype(_BF16)

    def l6(a, rows):
        h = _lrelu(jnp.dot(a, supb_ref[:_N, :32], preferred_element_type=_F32) + b6_ref[...])
        supa_ref[rows, :64] = jnp.dot(h, w7_ref[...], preferred_element_type=_F32).astype(_BF16)

    def l7(a, rows):
        h = _lrelu(jnp.dot(a, supa_ref[:_N, :64], preferred_element_type=_F32) + b7_ref[...])
        supb_ref[rows, :] = jnp.dot(h, w8_ref[...], preferred_element_type=_F32).astype(_BF16)

    def l8(a, rows, out_blk_ref):
        del rows
        out_blk_ref[...] = jnp.dot(a, supb_ref[:_N, :], preferred_element_type=_F32) + b8_ref[...]

    lat_spec = pl.BlockSpec((_TM, 8), lambda i: (i, 0))
    out_spec = pl.BlockSpec((_TM, 128), lambda i: (i, 0))
    run_layer(l2)
    run_layer(l3)
    run_layer(l4, (lat_spec,), (lat_ref,))
    run_layer(l5)
    run_layer(l6)
    run_layer(l7)
    run_layer(l8, (out_spec,), (out_ref,))


def _row_spec(tm, d):
    return pl.BlockSpec((tm, d), lambda i: (i, 0))


def _full_spec(r, c):
    return pl.BlockSpec((r, c), lambda i: (0, 0))


def kernel(x, adj, inv_adj, W1, b1, W2, b2, W3, b3, W4, b4, W5, b5, W6, b6,
           W7, b7, W8, b8):
    del inv_adj
    n, d0 = x.shape
    bs = [b.reshape(1, -1) for b in (b1, b2, b3, b4, b5, b6, b7, b8)]

    sup1 = pl.pallas_call(
        _sup1_body,
        grid=(pl.cdiv(n, 800),),
        in_specs=[_row_spec(800, d0), _full_spec(d0, 64)],
        out_specs=_row_spec(800, 64),
        out_shape=jax.ShapeDtypeStruct((n, 64), _BF16),
        compiler_params=_PARAMS,
    )(x, W1)

    adj16, sup2 = pl.pallas_call(
        _layer1_body,
        grid=(n // _TM1,),
        in_specs=[
            _row_spec(_TM1, n),
            _full_spec(n, 64),
            _full_spec(64, 32),
            _full_spec(1, 64),
        ],
        out_specs=[_row_spec(_TM1, n), _row_spec(_TM1, 32)],
        out_shape=[
            jax.ShapeDtypeStruct((n, n), _BF16),
            jax.ShapeDtypeStruct((n, 32), _BF16),
        ],
        compiler_params=_PARAMS,
    )(adj, sup1, W2, bs[0])

    vmem = pl.BlockSpec(memory_space=pltpu.VMEM)
    lat, out = pl.pallas_call(
        _deep_body,
        in_specs=[pl.BlockSpec(memory_space=pl.ANY)] + [vmem] * 14,
        out_specs=[pl.BlockSpec(memory_space=pl.ANY)] * 2,
        out_shape=[
            jax.ShapeDtypeStruct((n, 8), _F32),
            jax.ShapeDtypeStruct((n, 128), _F32),
        ],
        scratch_shapes=[
            pltpu.VMEM((_NR, 64), _BF16),
            pltpu.VMEM((_NR, 128), _BF16),
            pltpu.SMEM((1,), jnp.int32),
        ],
        compiler_params=_PARAMS,
    )(adj16, sup2, W3, W4, W5, W6, W7, W8, *bs[1:])

    return (lat, out)


# speedup vs baseline: 1.0373x; 1.0373x over previous
import jax
import jax.numpy as jnp
from jax.experimental import pallas as pl
from jax.experimental.pallas import tpu as pltpu

_N = 10000
_TM1 = 400
_TM = 400
_NBLK = _N // _TM
_NR = _N
_F32 = jnp.float32
_BF16 = jnp.bfloat16
_PARAMS = pltpu.CompilerParams(vmem_limit_bytes=120 * 1024 * 1024)

_STREAM_SPEC = pl.BlockSpec(
    (_TM, _N), lambda i: (i, 0), pipeline_mode=pl.Buffered(buffer_count=4)
)


def _lrelu(y):
    return jnp.where(y > 0, y, 0.01 * y)


def _sup1_body(x_ref, w_ref, o_ref):
    o_ref[...] = jnp.dot(
        x_ref[...], w_ref[...], preferred_element_type=_F32
    ).astype(_BF16)


def _layer1_body(a_ref, s_ref, w_ref, b_ref, a16_ref, sup_ref):
    a16 = a_ref[...].astype(_BF16)
    a16_ref[...] = a16
    y = jnp.dot(a16, s_ref[...], preferred_element_type=_F32) + b_ref[...]
    h = _lrelu(y)
    sup_ref[...] = jnp.dot(h, w_ref[...], preferred_element_type=_F32).astype(_BF16)


def _deep_body(adj_ref, s2_ref, w3_ref, w4_ref, w5_ref, w6_ref, w7_ref, w8_ref,
               b2_ref, b3_ref, b4_ref, b5_ref, b6_ref, b7_ref, b8_ref,
               lat_ref, out_ref, supa_ref, supb_ref, cnt_ref):

    def run_layer(step, out_specs=(), out_refs=()):
        cnt_ref[0] = 0

        def inner(a_ref, *orefs):
            i = cnt_ref[0]
            cnt_ref[0] = i + 1
            step(a_ref[...], pl.ds(i * _TM, _TM), *orefs)

        pltpu.emit_pipeline(
            inner, grid=(_NBLK,), in_specs=[_STREAM_SPEC],
            out_specs=list(out_specs),
        )(adj_ref, *out_refs)

    def l2(a, rows):
        h = _lrelu(jnp.dot(a, s2_ref[...], preferred_element_type=_F32) + b2_ref[...])
        supa_ref[rows, :16] = jnp.dot(h, w3_ref[...], preferred_element_type=_F32).astype(_BF16)

    def l3(a, rows):
        h = _lrelu(jnp.dot(a, supa_ref[:_N, :16], preferred_element_type=_F32) + b3_ref[...])
        supb_ref[rows, :8] = jnp.dot(h, w4_ref[...], preferred_element_type=_F32).astype(_BF16)

    def l4(a, rows, lat_blk_ref):
        y = jnp.dot(a, supb_ref[:_N, :8], preferred_element_type=_F32) + b4_ref[...]
        lat_blk_ref[...] = y
        supa_ref[rows, :16] = jnp.dot(y, w5_ref[...], preferred_element_type=_F32).astype(_BF16)

    def l5(a, rows):
        h = _lrelu(jnp.dot(a, supa_ref[:_N, :16], preferred_element_type=_F32) + b5_ref[...])
        supb_ref[rows, :32] = jnp.dot(h, w6_ref[...], preferred_element_type=_F32).astype(_BF16)

    def l6(a, rows):
        h = _lrelu(jnp.dot(a, supb_ref[:_N, :32], preferred_element_type=_F32) + b6_ref[...])
        supa_ref[rows, :64] = jnp.dot(h, w7_ref[...], preferred_element_type=_F32).astype(_BF16)

    def l7(a, rows):
        h = _lrelu(jnp.dot(a, supa_ref[:_N, :64], preferred_element_type=_F32) + b7_ref[...])
        supb_ref[rows, :] = jnp.dot(h, w8_ref[...], preferred_element_type=_F32).astype(_BF16)

    def l8(a, rows, out_blk_ref):
        del rows
        out_blk_ref[...] = jnp.dot(a, supb_ref[:_N, :], preferred_element_type=_F32) + b8_ref[...]

    lat_spec = pl.BlockSpec((_TM, 8), lambda i: (i, 0))
    out_spec = pl.BlockSpec((_TM, 128), lambda i: (i, 0))
    run_layer(l2)
    run_layer(l3)
    run_layer(l4, (lat_spec,), (lat_ref,))
    run_layer(l5)
    run_layer(l6)
    run_layer(l7)
    run_layer(l8, (out_spec,), (out_ref,))


def _row_spec(tm, d):
    return pl.BlockSpec((tm, d), lambda i: (i, 0))


def _full_spec(r, c):
    return pl.BlockSpec((r, c), lambda i: (0, 0))


def kernel(x, adj, inv_adj, W1, b1, W2, b2, W3, b3, W4, b4, W5, b5, W6, b6,
           W7, b7, W8, b8):
    del inv_adj
    n, d0 = x.shape
    bs = [b.reshape(1, -1) for b in (b1, b2, b3, b4, b5, b6, b7, b8)]

    sup1 = pl.pallas_call(
        _sup1_body,
        grid=(pl.cdiv(n, 800),),
        in_specs=[_row_spec(800, d0), _full_spec(d0, 64)],
        out_specs=_row_spec(800, 64),
        out_shape=jax.ShapeDtypeStruct((n, 64), _BF16),
        compiler_params=_PARAMS,
    )(x, W1)

    adj16, sup2 = pl.pallas_call(
        _layer1_body,
        grid=(n // _TM1,),
        in_specs=[
            _row_spec(_TM1, n),
            _full_spec(n, 64),
            _full_spec(64, 32),
            _full_spec(1, 64),
        ],
        out_specs=[_row_spec(_TM1, n), _row_spec(_TM1, 32)],
        out_shape=[
            jax.ShapeDtypeStruct((n, n), _BF16),
            jax.ShapeDtypeStruct((n, 32), _BF16),
        ],
        compiler_params=_PARAMS,
    )(adj, sup1, W2, bs[0])

    vmem = pl.BlockSpec(memory_space=pltpu.VMEM)
    lat, out = pl.pallas_call(
        _deep_body,
        in_specs=[pl.BlockSpec(memory_space=pl.ANY)] + [vmem] * 14,
        out_specs=[pl.BlockSpec(memory_space=pl.ANY)] * 2,
        out_shape=[
            jax.ShapeDtypeStruct((n, 8), _F32),
            jax.ShapeDtypeStruct((n, 128), _F32),
        ],
        scratch_shapes=[
            pltpu.VMEM((_NR, 64), _BF16),
            pltpu.VMEM((_NR, 128), _BF16),
            pltpu.SMEM((1,), jnp.int32),
        ],
        compiler_params=_PARAMS,
    )(adj16, sup2, W3, W4, W5, W6, W7, W8, *bs[1:])

    return (lat, out)
